# baseline (speedup 1.0000x reference)
import functools

import jax
import jax.numpy as jnp
from jax import lax
from jax.experimental import pallas as pl
from jax.experimental.pallas import tpu as pltpu
from jax.experimental.pallas import tpu_sc as plsc

B, T, M = 64, 1024, 128
H = 128
D, K = 64, 1024
N = B * T
BETA = 0.25

TB = 512
NBLK = N // TB

NW = 32
B_PER_W = N // NW
CH = 512


_DN = (((1,), (1,)), ((), ()))


def _mmbf(a, b):
    return lax.dot_general(a.astype(jnp.bfloat16), b.astype(jnp.bfloat16),
                           _DN, preferred_element_type=jnp.float32)


def _round_bf16(a):
    ai = lax.bitcast_convert_type(a, jnp.int32)
    r = ai + jnp.int32(0x7FFF) + lax.shift_right_logical(ai, 16) % 2
    r = lax.bitwise_and(r, jnp.int32(-65536))
    return lax.bitcast_convert_type(r, jnp.float32)


def _tc_body(x_ref, w1_ref, b1_ref, w2_ref, b2_ref, g_ref, bb_ref, cb_ref,
             cn_ref, z_ref, codes_ref, minsum_ref, counts_ref, perp_ref,
             cl_ref, cm_ref):
    i = pl.program_id(0)

    x = x_ref[...]
    h = _mmbf(x, w1_ref[...])
    h = h + b1_ref[...]
    h = 0.5 * h * (lax.erf(h * 0.7071067811865476) + 1.0)
    z = _mmbf(h, w2_ref[...])
    z = z + b2_ref[...]
    mu = jnp.mean(z, axis=-1, keepdims=True)
    var = jnp.mean((z - mu) ** 2, axis=-1, keepdims=True)
    z = (z - mu) / jnp.sqrt(var + 1e-5) * g_ref[...] + bb_ref[...]
    z_ref[...] = z

    row = jnp.sum(z * z, axis=-1, keepdims=True)
    s = lax.dot_general(z.astype(jnp.bfloat16),
                        cb_ref[...].astype(jnp.bfloat16),
                        (((1,), (1,)), ((), ())),
                        preferred_element_type=jnp.float32)
    dist = (row + cn_ref[...]) - 2.0 * s
    mn = jnp.min(dist, axis=-1, keepdims=True)
    iota = lax.broadcasted_iota(jnp.int32, (TB, K), 1)
    codes = jnp.min(jnp.where(dist == mn, iota, K), axis=-1,
                    keepdims=True)
    codes_ref[...] = codes
    onehot = jnp.where(iota == codes, 1.0, 0.0)
    cnt = jnp.sum(onehot, axis=0, keepdims=True)

    @pl.when(i == 0)
    def _init():
        minsum_ref[...] = jnp.zeros_like(minsum_ref)
        counts_ref[...] = jnp.zeros_like(counts_ref)

    msum = minsum_ref[...] + jnp.sum(mn).reshape(1, 1)
    ctot = counts_ref[...] + cnt
    minsum_ref[...] = msum
    counts_ref[...] = ctot

    @pl.when(i == NBLK - 1)
    def _finalize():
        avg = ctot * (1.0 / N)
        ent = -jnp.sum(avg * jnp.log(avg + 1e-10))
        perp_ref[...] = jnp.exp(ent).reshape(1, 1)
        cl = msum * (1.0 / (N * D))
        cl_ref[...] = cl
        cm_ref[...] = BETA * cl

    @pl.when(i != NBLK - 1)
    def _placeholder():
        perp_ref[...] = jnp.zeros_like(perp_ref)
        cl_ref[...] = jnp.zeros_like(cl_ref)
        cm_ref[...] = jnp.zeros_like(cm_ref)


def _tc_call(x2d, W1, b1, W2, b2, ln_g, ln_b, codebook, c_norm,
             interpret=False):
    const = lambda shape: pl.BlockSpec(shape, lambda i: (0, 0))
    return pl.pallas_call(
        _tc_body,
        grid=(NBLK,),
        in_specs=[
            pl.BlockSpec((TB, M), lambda i: (i, 0)),
            const((H, M)), const((1, H)), const((D, H)), const((1, D)),
            const((1, D)), const((1, D)), const((K, D)), const((1, K)),
        ],
        out_specs=[
            pl.BlockSpec((TB, D), lambda i: (i, 0)),
            pl.BlockSpec((TB, 1), lambda i: (i, 0)),
            const((1, 1)), const((1, K)), const((1, 1)), const((1, 1)),
            const((1, 1)),
        ],
        out_shape=[
            jax.ShapeDtypeStruct((N, D), jnp.float32),
            jax.ShapeDtypeStruct((N, 1), jnp.int32),
            jax.ShapeDtypeStruct((1, 1), jnp.float32),
            jax.ShapeDtypeStruct((1, K), jnp.float32),
            jax.ShapeDtypeStruct((1, 1), jnp.float32),
            jax.ShapeDtypeStruct((1, 1), jnp.float32),
            jax.ShapeDtypeStruct((1, 1), jnp.float32),
        ],
        interpret=interpret,
    )(x2d, W1, b1.reshape(1, H), W2, b2.reshape(1, D), ln_g.reshape(1, D),
      ln_b.reshape(1, D), codebook, c_norm)


def _sc_gather(codebook, codes_flat):
    mesh = plsc.VectorSubcoreMesh(core_axis_name="c", subcore_axis_name="s")

    @functools.partial(
        pl.kernel, mesh=mesh,
        compiler_params=pltpu.CompilerParams(use_tc_tiling_on_sc=False),
        out_type=jax.ShapeDtypeStruct((N, D), jnp.float32),
        scratch_types=[
            pltpu.VMEM((CH,), jnp.int32),
            pltpu.VMEM((CH, D), jnp.float32),
            pltpu.SemaphoreType.DMA,
        ],
    )
    def gather_k(cb_hbm, codes_hbm, out_hbm, idx_v, rows_v, sem):
        wid = lax.axis_index("s") * 2 + lax.axis_index("c")
        base = wid * B_PER_W
        for c in range(B_PER_W // CH):
            off = base + c * CH
            pltpu.sync_copy(codes_hbm.at[pl.ds(off, CH)], idx_v)
            pltpu.async_copy(cb_hbm.at[idx_v], rows_v, sem).wait()
            pltpu.sync_copy(rows_v, out_hbm.at[pl.ds(off, CH)])

    return gather_k(codebook, codes_flat)


def kernel(x, W1, b1, W2, b2, ln_g, ln_b, codebook):
    x2d = x.reshape(N, M)
    c_norm = (codebook ** 2).sum(1)[None, :]
    z_e2d, codes2d, _minsum, _counts, perp, cl, cm = _tc_call(
        x2d, W1, b1, W2, b2, ln_g, ln_b, codebook, c_norm)
    codes_flat = codes2d.reshape(N)
    z_q2d = _sc_gather(codebook, codes_flat)
    z_q_st = z_e2d + lax.stop_gradient(z_q2d - z_e2d)
    return (z_e2d.reshape(B, T, D), z_q_st.reshape(B, T, D),
            codes_flat.reshape(B, T), perp.reshape(()), cl.reshape(()),
            cm.reshape(()))


# speedup vs baseline: 1.1071x; 1.1071x over previous
import functools

import jax
import jax.numpy as jnp
from jax import lax
from jax.experimental import pallas as pl
from jax.experimental.pallas import tpu as pltpu
from jax.experimental.pallas import tpu_sc as plsc

B, T, M = 64, 1024, 128
H = 128
D, K = 64, 1024
N = B * T
BETA = 0.25

TB = 1024
NBLK = N // TB

NW = 32
B_PER_W = N // NW
CH = 512


_DN = (((1,), (1,)), ((), ()))


def _mmbf(a, b):
    return lax.dot_general(a.astype(jnp.bfloat16), b.astype(jnp.bfloat16),
                           _DN, preferred_element_type=jnp.float32)


def _round_bf16(a):
    ai = lax.bitcast_convert_type(a, jnp.int32)
    r = ai + jnp.int32(0x7FFF) + lax.shift_right_logical(ai, 16) % 2
    r = lax.bitwise_and(r, jnp.int32(-65536))
    return lax.bitcast_convert_type(r, jnp.float32)


def _tc_body(x_ref, w1_ref, b1_ref, w2_ref, b2_ref, g_ref, bb_ref, cb_ref,
             cn_ref, z_ref, codes_ref, minsum_ref, counts_ref, perp_ref,
             cl_ref, cm_ref):
    i = pl.program_id(0)

    x = x_ref[...]
    h = _mmbf(x, w1_ref[...])
    h = h + b1_ref[...]
    h = 0.5 * h * (lax.erf(h * 0.7071067811865476) + 1.0)
    z = _mmbf(h, w2_ref[...])
    z = z + b2_ref[...]
    mu = jnp.mean(z, axis=-1, keepdims=True)
    var = jnp.mean((z - mu) ** 2, axis=-1, keepdims=True)
    z = (z - mu) / jnp.sqrt(var + 1e-5) * g_ref[...] + bb_ref[...]
    z_ref[...] = z

    row = jnp.sum(z * z, axis=-1, keepdims=True)
    s = lax.dot_general(z.astype(jnp.bfloat16),
                        cb_ref[...].astype(jnp.bfloat16),
                        (((1,), (1,)), ((), ())),
                        preferred_element_type=jnp.float32)
    dist = (row + cn_ref[...]) - 2.0 * s
    mn = jnp.min(dist, axis=-1, keepdims=True)
    iota = lax.broadcasted_iota(jnp.int32, (TB, K), 1)
    codes = jnp.min(jnp.where(dist == mn, iota, K), axis=-1,
                    keepdims=True)
    codes_ref[...] = codes
    onehot = jnp.where(iota == codes, 1.0, 0.0)
    cnt = jnp.sum(onehot, axis=0, keepdims=True)

    @pl.when(i == 0)
    def _init():
        minsum_ref[...] = jnp.zeros_like(minsum_ref)
        counts_ref[...] = jnp.zeros_like(counts_ref)

    msum = minsum_ref[...] + jnp.sum(mn).reshape(1, 1)
    ctot = counts_ref[...] + cnt
    minsum_ref[...] = msum
    counts_ref[...] = ctot

    @pl.when(i == NBLK - 1)
    def _finalize():
        avg = ctot * (1.0 / N)
        ent = -jnp.sum(avg * jnp.log(avg + 1e-10))
        perp_ref[...] = jnp.exp(ent).reshape(1, 1)
        cl = msum * (1.0 / (N * D))
        cl_ref[...] = cl
        cm_ref[...] = BETA * cl

    @pl.when(i != NBLK - 1)
    def _placeholder():
        perp_ref[...] = jnp.zeros_like(perp_ref)
        cl_ref[...] = jnp.zeros_like(cl_ref)
        cm_ref[...] = jnp.zeros_like(cm_ref)


def _tc_call(x2d, W1, b1, W2, b2, ln_g, ln_b, codebook, c_norm,
             interpret=False):
    const = lambda shape: pl.BlockSpec(shape, lambda i: (0, 0))
    return pl.pallas_call(
        _tc_body,
        grid=(NBLK,),
        in_specs=[
            pl.BlockSpec((TB, M), lambda i: (i, 0)),
            const((H, M)), const((1, H)), const((D, H)), const((1, D)),
            const((1, D)), const((1, D)), const((K, D)), const((1, K)),
        ],
        out_specs=[
            pl.BlockSpec((TB, D), lambda i: (i, 0)),
            pl.BlockSpec((TB, 1), lambda i: (i, 0)),
            const((1, 1)), const((1, K)), const((1, 1)), const((1, 1)),
            const((1, 1)),
        ],
        out_shape=[
            jax.ShapeDtypeStruct((N, D), jnp.float32),
            jax.ShapeDtypeStruct((N, 1), jnp.int32),
            jax.ShapeDtypeStruct((1, 1), jnp.float32),
            jax.ShapeDtypeStruct((1, K), jnp.float32),
            jax.ShapeDtypeStruct((1, 1), jnp.float32),
            jax.ShapeDtypeStruct((1, 1), jnp.float32),
            jax.ShapeDtypeStruct((1, 1), jnp.float32),
        ],
        interpret=interpret,
    )(x2d, W1, b1.reshape(1, H), W2, b2.reshape(1, D), ln_g.reshape(1, D),
      ln_b.reshape(1, D), codebook, c_norm)


def _sc_gather(codebook, codes_flat):
    mesh = plsc.VectorSubcoreMesh(core_axis_name="c", subcore_axis_name="s")

    @functools.partial(
        pl.kernel, mesh=mesh,
        compiler_params=pltpu.CompilerParams(use_tc_tiling_on_sc=False),
        out_type=jax.ShapeDtypeStruct((N, D), jnp.float32),
        scratch_types=[
            pltpu.VMEM((CH,), jnp.int32),
            pltpu.VMEM((CH, D), jnp.float32),
            pltpu.SemaphoreType.DMA,
        ],
    )
    def gather_k(cb_hbm, codes_hbm, out_hbm, idx_v, rows_v, sem):
        wid = lax.axis_index("s") * 2 + lax.axis_index("c")
        base = wid * B_PER_W
        for c in range(B_PER_W // CH):
            off = base + c * CH
            pltpu.sync_copy(codes_hbm.at[pl.ds(off, CH)], idx_v)
            pltpu.async_copy(cb_hbm.at[idx_v], rows_v, sem).wait()
            pltpu.sync_copy(rows_v, out_hbm.at[pl.ds(off, CH)])

    return gather_k(codebook, codes_flat)


def kernel(x, W1, b1, W2, b2, ln_g, ln_b, codebook):
    x2d = x.reshape(N, M)
    c_norm = (codebook ** 2).sum(1)[None, :]
    z_e2d, codes2d, _minsum, _counts, perp, cl, cm = _tc_call(
        x2d, W1, b1, W2, b2, ln_g, ln_b, codebook, c_norm)
    codes_flat = codes2d.reshape(N)
    z_q2d = _sc_gather(codebook, codes_flat)
    z_q_st = z_e2d + lax.stop_gradient(z_q2d - z_e2d)
    return (z_e2d.reshape(B, T, D), z_q_st.reshape(B, T, D),
            codes_flat.reshape(B, T), perp.reshape(()), cl.reshape(()),
            cm.reshape(()))
